# baseline (speedup 1.0000x reference)
import functools

import jax
import jax.numpy as jnp
from jax import lax
from jax.experimental import pallas as pl
from jax.experimental.pallas import tpu as pltpu
from jax.experimental.pallas import tpu_sc as plsc

V, D = 1000000, 64
B = 16384
H = V // 2
PL = 2 * D

_info = plsc.get_sparse_core_info()
NC, NS = _info.num_cores, _info.num_subcores
NW = NC * NS
BPW = B // NW
CHUNK = 128
C = BPW // CHUNK
L = 16

_mesh = plsc.VectorSubcoreMesh(core_axis_name="c", subcore_axis_name="s")


@functools.partial(
    pl.kernel,
    mesh=_mesh,
    out_type=jax.ShapeDtypeStruct((B, D), jnp.float32),
    scratch_types=[
        pltpu.VMEM((C, CHUNK), jnp.int32),
        pltpu.VMEM((BPW,), jnp.int32),
        pltpu.VMEM((2, CHUNK, PL), jnp.float32),
        pltpu.VMEM((BPW, D), jnp.float32),
        pltpu.SemaphoreType.DMA,
        pltpu.SemaphoreType.DMA,
    ],
    compiler_params=pltpu.CompilerParams(needs_layout_passes=False),
)
def _gather_sc(x2_hbm, lid_hbm, hof_hbm, out_hbm, lid_v, hof_v, grp_v, rows_v,
               sem0, sem1):
    wid = lax.axis_index("s") * NC + lax.axis_index("c")
    base = wid * BPW
    pltpu.sync_copy(lid_hbm.at[wid], lid_v)
    pltpu.sync_copy(hof_hbm.at[pl.ds(base, BPW)], hof_v)
    sems = [sem0, sem1]

    def extract(j):
        b = j % 2

        def group(g, carry):
            i0 = j * CHUNK + g * L
            k_vec = g * L + lax.iota(jnp.int32, L)
            h_vec = hof_v[pl.ds(i0, L)]
            ko_vec = i0 + lax.iota(jnp.int32, L)
            for c in range(D):
                vals = plsc.load_gather(grp_v.at[b], [k_vec, h_vec + c])
                plsc.store_scatter(
                    rows_v, [ko_vec, jnp.full((L,), c, jnp.int32)], vals
                )
            return carry

        lax.fori_loop(0, CHUNK // L, group, 0)

    copies = [None, None]
    for j in range(C):
        b = j % 2
        copies[b] = pltpu.async_copy(
            x2_hbm.at[lid_v.at[j]], grp_v.at[b], sems[b]
        )
        if j >= 1:
            copies[(j - 1) % 2].wait()
            extract(j - 1)
    copies[(C - 1) % 2].wait()
    extract(C - 1)
    pltpu.sync_copy(rows_v, out_hbm.at[pl.ds(base, BPW)])


def kernel(x, index):
    x2 = jnp.concatenate([x[:H], x[H:]], axis=1)
    hi = (index >= H).astype(jnp.int32)
    lid = (index - hi * H).reshape(NW, C, CHUNK)
    hof = hi * D
    return _gather_sc(x2, lid, hof)


# speedup vs baseline: 2.1881x; 2.1881x over previous
import functools

import jax
import jax.numpy as jnp
from jax import lax
from jax.experimental import pallas as pl
from jax.experimental.pallas import tpu as pltpu
from jax.experimental.pallas import tpu_sc as plsc

V, D = 1000000, 64
B = 16384

_info = plsc.get_sparse_core_info()
NC, NS = _info.num_cores, _info.num_subcores
NW = NC * NS
BPW = B // NW
K = 16

_mesh = plsc.VectorSubcoreMesh(core_axis_name="c", subcore_axis_name="s")


@functools.partial(
    pl.kernel,
    mesh=_mesh,
    out_type=jax.ShapeDtypeStruct((B, D), jnp.float32),
    scratch_types=[
        pltpu.VMEM((BPW,), jnp.int32),
        pltpu.VMEM((BPW, D), jnp.float32),
        pltpu.SemaphoreType.DMA,
    ],
)
def _gather_sc(x_hbm, idx_hbm, out_hbm, idx_v, rows_v, sem):
    wid = lax.axis_index("s") * NC + lax.axis_index("c")
    base = wid * BPW
    pltpu.sync_copy(idx_hbm.at[pl.ds(base, BPW)], idx_v)

    def burst(j, carry):
        i0 = j * K
        idx_vec = idx_v[pl.ds(i0, K)]
        for t in range(K):
            r = idx_vec[t]
            pltpu.async_copy(x_hbm.at[r], rows_v.at[i0 + t], sem)
        return carry

    lax.fori_loop(0, BPW // K, burst, 0)
    pltpu.make_async_copy(x_hbm.at[pl.ds(0, BPW)], rows_v, sem).wait()
    pltpu.sync_copy(rows_v, out_hbm.at[pl.ds(base, BPW)])


def kernel(x, index):
    return _gather_sc(x, index)
